# baseline (speedup 1.0000x reference)
import jax
import jax.numpy as jnp
from jax import lax
from jax.experimental import pallas as pl
from jax.experimental.pallas import tpu as pltpu
from jax.experimental.pallas import tpu_sc as plsc

_N = 10000
_F = 128
_NPAD = 10240
_E = 320000
_NTILES = 32
_CHUNK = 112
_NBUF = 3
_S0 = 25
_S1 = 5
_NCHTOT = 6 * (_S0 + _S1) * 16
_EPAD = _NCHTOT * _CHUNK
_ROWS_PT = _NPAD // 16
_NEG_ATT = 0.2
_NEG_ACT = 0.01
_EPS = 1e-16


def _sc_edge_body(src_hbm, dst_hbm, asrc_hbm, adst_hbm, h_hbm,
                  acc_out, den_out,
                  idx_s, idx_d, sv, dv, wv, rows,
                  isem, lsem, gsem, ssem, dsem, acc_sh, den_sh):
    cid = lax.axis_index("c")
    sid = lax.axis_index("s")
    nsuper = jnp.where(cid == 0, _S0, _S1)
    chunk0 = jnp.where(cid == 0, sid * (6 * _S0),
                       16 * 6 * _S0 + sid * (6 * _S1))

    zero16 = jnp.zeros((16,), jnp.float32)

    def _zrow(i, carry):
        for j in range(_F // 16):
            rows[0, i, pl.ds(j * 16, 16)] = zero16
        return carry

    lax.fori_loop(0, _CHUNK, _zrow, 0)
    for g in range(_CHUNK // 16):
        wv[0, pl.ds(g * 16, 16)] = zero16
    row0 = sid * _ROWS_PT
    for k in range(_ROWS_PT // _CHUNK):
        pltpu.sync_copy(rows.at[0], acc_sh.at[pl.ds(row0 + k * _CHUNK,
                                                    _CHUNK)])
        pltpu.sync_copy(wv.at[0], den_sh.at[pl.ds(row0 + k * _CHUNK, _CHUNK)])
    rem = _ROWS_PT % _CHUNK
    if rem:
        tail = row0 + (_ROWS_PT // _CHUNK) * _CHUNK
        pltpu.sync_copy(rows.at[0, pl.ds(0, rem)],
                        acc_sh.at[pl.ds(tail, rem)])
        pltpu.sync_copy(wv.at[0, pl.ds(0, rem)], den_sh.at[pl.ds(tail, rem)])
    plsc.subcore_barrier()

    def _issue_idx(i, b, islot):
        pltpu.async_copy(src_hbm.at[chunk0 + i], idx_s.at[islot], isem[b])
        pltpu.async_copy(dst_hbm.at[chunk0 + i], idx_d.at[islot], isem[b])

    def _wait_idx(i, b, islot):
        pltpu.make_async_copy(src_hbm.at[chunk0 + i], idx_s.at[islot],
                              isem[b]).wait()
        pltpu.make_async_copy(dst_hbm.at[chunk0 + i], idx_d.at[islot],
                              isem[b]).wait()

    def _issue_logits(b, islot):
        pltpu.async_copy(asrc_hbm.at[idx_s.at[islot]], sv.at[b], lsem[b])
        pltpu.async_copy(adst_hbm.at[idx_d.at[islot]], dv.at[b], lsem[b])

    def _wait_logits(b, islot):
        pltpu.make_async_copy(asrc_hbm.at[idx_s.at[islot]], sv.at[b],
                              lsem[b]).wait()
        pltpu.make_async_copy(adst_hbm.at[idx_d.at[islot]], dv.at[b],
                              lsem[b]).wait()

    def _issue_rows(b, islot):
        pltpu.async_copy(h_hbm.at[idx_s.at[islot]], rows.at[b], gsem[b])

    def _wait_rows(b, islot):
        pltpu.make_async_copy(h_hbm.at[idx_s.at[islot]], rows.at[b],
                              gsem[b]).wait()

    def _issue_scat_rows(b, islot):
        pltpu.async_copy(rows.at[b], acc_sh.at[idx_d.at[islot]], ssem[b],
                         add=True)

    def _wait_scat_rows(b, islot):
        pltpu.make_async_copy(rows.at[b], acc_sh.at[idx_d.at[islot]],
                              ssem[b]).wait()

    def _issue_scat_den(b, islot):
        pltpu.async_copy(wv.at[b], den_sh.at[idx_d.at[islot]], dsem[b],
                         add=True)

    def _wait_scat_den(b, islot):
        pltpu.make_async_copy(wv.at[b], den_sh.at[idx_d.at[islot]],
                              dsem[b]).wait()

    def _process(b, islot):
        _wait_logits(b, islot)
        for g in range(_CHUNK // 16):
            e16 = sv[b, pl.ds(g * 16, 16)] + dv[b, pl.ds(g * 16, 16)]
            e16 = jnp.maximum(e16, _NEG_ATT * e16)
            wv[b, pl.ds(g * 16, 16)] = jnp.exp(e16)
        _issue_scat_den(b, islot)
        _wait_rows(b, islot)
        b16 = jnp.full((16,), b, jnp.int32)

        @plsc.parallel_loop(0, _CHUNK, unroll=2)
        def _scale(e):
            w16 = plsc.load_gather(wv, [b16, jnp.full((16,), e, jnp.int32)])
            for j in range(_F // 16):
                rows[b, e, pl.ds(j * 16, 16)] = (
                    rows[b, e, pl.ds(j * 16, 16)] * w16)

        _issue_scat_rows(b, islot)

    for b in range(_NBUF):
        pltpu.sync_copy(src_hbm.at[chunk0 + b], idx_s.at[b])
        pltpu.sync_copy(dst_hbm.at[chunk0 + b], idx_d.at[b])
    for b in range(_NBUF):
        _issue_idx(_NBUF + b, b, _NBUF + b)
        _issue_logits(b, b)
        _issue_rows(b, b)

    def _super(s, carry):
        for r in range(2):
            base = 6 * s + 3 * r

            def _top():
                for b in range(_NBUF):
                    islot = 3 * r + b
                    islot_n = (islot + 3) % 6
                    _wait_scat_rows(b, islot_n)
                    _wait_scat_den(b, islot_n)
                    _wait_idx(base + b, b, islot)
                    _issue_logits(b, islot)
                    _issue_rows(b, islot)

            def _issue_next_idx():
                for b in range(_NBUF):
                    islot_n = (3 * r + b + 3) % 6
                    _issue_idx(base + b + 3, b, islot_n)

            if r == 0:
                @pl.when(s > 0)
                def _():
                    _top()
                    _issue_next_idx()
            else:
                _top()

                @pl.when(s + 1 < nsuper)
                def _():
                    _issue_next_idx()

            for b in range(_NBUF):
                _process(b, 3 * r + b)
        return carry

    lax.fori_loop(0, nsuper, _super, 0)

    for b in range(_NBUF):
        _wait_scat_rows(b, 3 + b)
        _wait_scat_den(b, 3 + b)

    plsc.subcore_barrier()
    pltpu.sync_copy(acc_sh.at[pl.ds(row0, _ROWS_PT)],
                    acc_out.at[cid, pl.ds(row0, _ROWS_PT)])
    pltpu.sync_copy(den_sh.at[pl.ds(row0, _ROWS_PT)],
                    den_out.at[cid, pl.ds(row0, _ROWS_PT)])


def _sc_edge(src2d, dst2d, a_src, a_dst, h):
    kern = pl.kernel(
        _sc_edge_body,
        out_type=(jax.ShapeDtypeStruct((2, _NPAD, _F), jnp.float32),
                  jax.ShapeDtypeStruct((2, _NPAD), jnp.float32)),
        mesh=plsc.VectorSubcoreMesh(core_axis_name="c", subcore_axis_name="s"),
        compiler_params=pltpu.CompilerParams(needs_layout_passes=False),
        scratch_types=[
            pltpu.VMEM((2 * _NBUF, _CHUNK), jnp.int32),
            pltpu.VMEM((2 * _NBUF, _CHUNK), jnp.int32),
            pltpu.VMEM((_NBUF, _CHUNK), jnp.float32),
            pltpu.VMEM((_NBUF, _CHUNK), jnp.float32),
            pltpu.VMEM((_NBUF, _CHUNK), jnp.float32),
            pltpu.VMEM((_NBUF, _CHUNK, _F), jnp.float32),
            [pltpu.SemaphoreType.DMA] * _NBUF,
            [pltpu.SemaphoreType.DMA] * _NBUF,
            [pltpu.SemaphoreType.DMA] * _NBUF,
            [pltpu.SemaphoreType.DMA] * _NBUF,
            [pltpu.SemaphoreType.DMA] * _NBUF,
            pltpu.VMEM_SHARED((_NPAD, _F), jnp.float32),
            pltpu.VMEM_SHARED((_NPAD,), jnp.float32),
        ],
    )
    return kern(src2d, dst2d, a_src, a_dst, h)


def _tc_dense(x, W, att_s, att_d):
    def body(x_ref, w_ref, as_ref, ad_ref, h_ref, s_ref, d_ref):
        h = jnp.dot(x_ref[...], w_ref[...],
                    preferred_element_type=jnp.float32)
        zf = jnp.zeros((_NPAD - _N, _F), jnp.float32)
        h_ref[...] = jnp.concatenate([h, zf], axis=0)
        zv = jnp.zeros((_NPAD - _N,), jnp.float32)
        s_ref[...] = jnp.concatenate(
            [jnp.sum(h * as_ref[...][None, :], axis=1), zv])
        d_ref[...] = jnp.concatenate(
            [jnp.sum(h * ad_ref[...][None, :], axis=1), zv])

    return pl.pallas_call(
        body,
        out_shape=(jax.ShapeDtypeStruct((_NPAD, _F), jnp.float32),
                   jax.ShapeDtypeStruct((_NPAD,), jnp.float32),
                   jax.ShapeDtypeStruct((_NPAD,), jnp.float32)),
    )(x, W, att_s, att_d)


def _tc_combine(acc, den, b, W, att_s, att_d):
    def body(acc_ref, den_ref, b_ref, w_ref, as_ref, ad_ref,
             h_ref, s_ref, d_ref):
        a = acc_ref[0] + acc_ref[1]
        dn = den_ref[0] + den_ref[1]
        h1 = a[:_N] / (dn[:_N, None] + _EPS) + b_ref[...][None, :]
        h1 = jnp.maximum(h1, _NEG_ACT * h1)
        h2 = jnp.dot(h1, w_ref[...], preferred_element_type=jnp.float32)
        zf = jnp.zeros((_NPAD - _N, _F), jnp.float32)
        h_ref[...] = jnp.concatenate([h2, zf], axis=0)
        zv = jnp.zeros((_NPAD - _N,), jnp.float32)
        s_ref[...] = jnp.concatenate(
            [jnp.sum(h2 * as_ref[...][None, :], axis=1), zv])
        d_ref[...] = jnp.concatenate(
            [jnp.sum(h2 * ad_ref[...][None, :], axis=1), zv])

    return pl.pallas_call(
        body,
        out_shape=(jax.ShapeDtypeStruct((_NPAD, _F), jnp.float32),
                   jax.ShapeDtypeStruct((_NPAD,), jnp.float32),
                   jax.ShapeDtypeStruct((_NPAD,), jnp.float32)),
    )(acc, den, b, W, att_s, att_d)


def _tc_final(acc, den, b):
    def body(acc_ref, den_ref, b_ref, out_ref):
        a = acc_ref[0] + acc_ref[1]
        dn = den_ref[0] + den_ref[1]
        out_ref[...] = a[:_N] / (dn[:_N, None] + _EPS) + b_ref[...][None, :]

    return pl.pallas_call(
        body,
        out_shape=jax.ShapeDtypeStruct((_N, _F), jnp.float32),
    )(acc, den, b)


def kernel(x, edge_index, W1, att_src1, att_dst1, bias1,
           W2, att_src2, att_dst2, bias2):
    src = edge_index[0].astype(jnp.int32)
    dst = edge_index[1].astype(jnp.int32)
    pad = jnp.full((_EPAD - _E,), _NPAD - 1, jnp.int32)
    srcp = jnp.concatenate([src, pad]).reshape(_NCHTOT, _CHUNK)
    dstp = jnp.concatenate([dst, pad]).reshape(_NCHTOT, _CHUNK)

    h1, s1, d1 = _tc_dense(x, W1, att_src1, att_dst1)
    acc1, den1 = _sc_edge(srcp, dstp, s1, d1, h1)
    h2, s2, d2 = _tc_combine(acc1, den1, bias1, W2, att_src2, att_dst2)
    acc2, den2 = _sc_edge(srcp, dstp, s2, d2, h2)
    return _tc_final(acc2, den2, bias2)


# speedup vs baseline: 1.0304x; 1.0304x over previous
import jax
import jax.numpy as jnp
from jax import lax
from jax.experimental import pallas as pl
from jax.experimental.pallas import tpu as pltpu
from jax.experimental.pallas import tpu_sc as plsc

_N = 10000
_F = 128
_NPAD = 10240
_E = 320000
_NTILES = 32
_CHUNK = 112
_NBUF = 3
_S0 = 24
_S1 = 6
_NCHTOT = 6 * (_S0 + _S1) * 16
_EPAD = _NCHTOT * _CHUNK
_ROWS_PT = _NPAD // 16
_NEG_ATT = 0.2
_NEG_ACT = 0.01
_EPS = 1e-16


def _sc_edge_body(src_hbm, dst_hbm, asrc_hbm, adst_hbm, h_hbm,
                  acc_out, den_out,
                  idx_s, idx_d, sv, dv, wv, rows,
                  isem, lsem, gsem, ssem, dsem, acc_sh, den_sh):
    cid = lax.axis_index("c")
    sid = lax.axis_index("s")
    nsuper = jnp.where(cid == 0, _S0, _S1)
    chunk0 = jnp.where(cid == 0, sid * (6 * _S0),
                       16 * 6 * _S0 + sid * (6 * _S1))

    zero16 = jnp.zeros((16,), jnp.float32)

    def _zrow(i, carry):
        for j in range(_F // 16):
            rows[0, i, pl.ds(j * 16, 16)] = zero16
        return carry

    lax.fori_loop(0, _CHUNK, _zrow, 0)
    for g in range(_CHUNK // 16):
        wv[0, pl.ds(g * 16, 16)] = zero16
    row0 = sid * _ROWS_PT
    for k in range(_ROWS_PT // _CHUNK):
        pltpu.sync_copy(rows.at[0], acc_sh.at[pl.ds(row0 + k * _CHUNK,
                                                    _CHUNK)])
        pltpu.sync_copy(wv.at[0], den_sh.at[pl.ds(row0 + k * _CHUNK, _CHUNK)])
    rem = _ROWS_PT % _CHUNK
    if rem:
        tail = row0 + (_ROWS_PT // _CHUNK) * _CHUNK
        pltpu.sync_copy(rows.at[0, pl.ds(0, rem)],
                        acc_sh.at[pl.ds(tail, rem)])
        pltpu.sync_copy(wv.at[0, pl.ds(0, rem)], den_sh.at[pl.ds(tail, rem)])
    plsc.subcore_barrier()

    def _issue_idx(i, b, islot):
        pltpu.async_copy(src_hbm.at[chunk0 + i], idx_s.at[islot], isem[b])
        pltpu.async_copy(dst_hbm.at[chunk0 + i], idx_d.at[islot], isem[b])

    def _wait_idx(i, b, islot):
        pltpu.make_async_copy(src_hbm.at[chunk0 + i], idx_s.at[islot],
                              isem[b]).wait()
        pltpu.make_async_copy(dst_hbm.at[chunk0 + i], idx_d.at[islot],
                              isem[b]).wait()

    def _issue_logits(b, islot):
        pltpu.async_copy(asrc_hbm.at[idx_s.at[islot]], sv.at[b], lsem[b])
        pltpu.async_copy(adst_hbm.at[idx_d.at[islot]], dv.at[b], lsem[b])

    def _wait_logits(b, islot):
        pltpu.make_async_copy(asrc_hbm.at[idx_s.at[islot]], sv.at[b],
                              lsem[b]).wait()
        pltpu.make_async_copy(adst_hbm.at[idx_d.at[islot]], dv.at[b],
                              lsem[b]).wait()

    def _issue_rows(b, islot):
        pltpu.async_copy(h_hbm.at[idx_s.at[islot]], rows.at[b], gsem[b])

    def _wait_rows(b, islot):
        pltpu.make_async_copy(h_hbm.at[idx_s.at[islot]], rows.at[b],
                              gsem[b]).wait()

    def _issue_scat_rows(b, islot):
        pltpu.async_copy(rows.at[b], acc_sh.at[idx_d.at[islot]], ssem[b],
                         add=True)

    def _wait_scat_rows(b, islot):
        pltpu.make_async_copy(rows.at[b], acc_sh.at[idx_d.at[islot]],
                              ssem[b]).wait()

    def _issue_scat_den(b, islot):
        pltpu.async_copy(wv.at[b], den_sh.at[idx_d.at[islot]], dsem[b],
                         add=True)

    def _wait_scat_den(b, islot):
        pltpu.make_async_copy(wv.at[b], den_sh.at[idx_d.at[islot]],
                              dsem[b]).wait()

    def _process(b, islot):
        _wait_logits(b, islot)
        for g in range(_CHUNK // 16):
            e16 = sv[b, pl.ds(g * 16, 16)] + dv[b, pl.ds(g * 16, 16)]
            e16 = jnp.maximum(e16, _NEG_ATT * e16)
            wv[b, pl.ds(g * 16, 16)] = jnp.exp(e16)
        _issue_scat_den(b, islot)
        _wait_rows(b, islot)
        b16 = jnp.full((16,), b, jnp.int32)

        @plsc.parallel_loop(0, _CHUNK, unroll=2)
        def _scale(e):
            w16 = plsc.load_gather(wv, [b16, jnp.full((16,), e, jnp.int32)])
            for j in range(_F // 16):
                rows[b, e, pl.ds(j * 16, 16)] = (
                    rows[b, e, pl.ds(j * 16, 16)] * w16)

        _issue_scat_rows(b, islot)

    for b in range(_NBUF):
        pltpu.sync_copy(src_hbm.at[chunk0 + b], idx_s.at[b])
        pltpu.sync_copy(dst_hbm.at[chunk0 + b], idx_d.at[b])
    for b in range(_NBUF):
        _issue_idx(_NBUF + b, b, _NBUF + b)
        _issue_logits(b, b)
        _issue_rows(b, b)

    def _super(s, carry):
        for r in range(2):
            base = 6 * s + 3 * r

            def _top():
                for b in range(_NBUF):
                    islot = 3 * r + b
                    islot_n = (islot + 3) % 6
                    _wait_scat_rows(b, islot_n)
                    _wait_scat_den(b, islot_n)
                    _wait_idx(base + b, b, islot)
                    _issue_logits(b, islot)
                    _issue_rows(b, islot)

            def _issue_next_idx():
                for b in range(_NBUF):
                    islot_n = (3 * r + b + 3) % 6
                    _issue_idx(base + b + 3, b, islot_n)

            if r == 0:
                @pl.when(s > 0)
                def _():
                    _top()
                    _issue_next_idx()
            else:
                _top()

                @pl.when(s + 1 < nsuper)
                def _():
                    _issue_next_idx()

            for b in range(_NBUF):
                _process(b, 3 * r + b)
        return carry

    lax.fori_loop(0, nsuper, _super, 0)

    for b in range(_NBUF):
        _wait_scat_rows(b, 3 + b)
        _wait_scat_den(b, 3 + b)

    plsc.subcore_barrier()
    pltpu.sync_copy(acc_sh.at[pl.ds(row0, _ROWS_PT)],
                    acc_out.at[cid, pl.ds(row0, _ROWS_PT)])
    pltpu.sync_copy(den_sh.at[pl.ds(row0, _ROWS_PT)],
                    den_out.at[cid, pl.ds(row0, _ROWS_PT)])


def _sc_edge(src2d, dst2d, a_src, a_dst, h):
    kern = pl.kernel(
        _sc_edge_body,
        out_type=(jax.ShapeDtypeStruct((2, _NPAD, _F), jnp.float32),
                  jax.ShapeDtypeStruct((2, _NPAD), jnp.float32)),
        mesh=plsc.VectorSubcoreMesh(core_axis_name="c", subcore_axis_name="s"),
        compiler_params=pltpu.CompilerParams(needs_layout_passes=False),
        scratch_types=[
            pltpu.VMEM((2 * _NBUF, _CHUNK), jnp.int32),
            pltpu.VMEM((2 * _NBUF, _CHUNK), jnp.int32),
            pltpu.VMEM((_NBUF, _CHUNK), jnp.float32),
            pltpu.VMEM((_NBUF, _CHUNK), jnp.float32),
            pltpu.VMEM((_NBUF, _CHUNK), jnp.float32),
            pltpu.VMEM((_NBUF, _CHUNK, _F), jnp.float32),
            [pltpu.SemaphoreType.DMA] * _NBUF,
            [pltpu.SemaphoreType.DMA] * _NBUF,
            [pltpu.SemaphoreType.DMA] * _NBUF,
            [pltpu.SemaphoreType.DMA] * _NBUF,
            [pltpu.SemaphoreType.DMA] * _NBUF,
            pltpu.VMEM_SHARED((_NPAD, _F), jnp.float32),
            pltpu.VMEM_SHARED((_NPAD,), jnp.float32),
        ],
    )
    return kern(src2d, dst2d, a_src, a_dst, h)


def _tc_dense(x, W, att_s, att_d):
    def body(x_ref, w_ref, as_ref, ad_ref, h_ref, s_ref, d_ref):
        h = jnp.dot(x_ref[...], w_ref[...],
                    preferred_element_type=jnp.float32)
        zf = jnp.zeros((_NPAD - _N, _F), jnp.float32)
        h_ref[...] = jnp.concatenate([h, zf], axis=0)
        zv = jnp.zeros((_NPAD - _N,), jnp.float32)
        s_ref[...] = jnp.concatenate(
            [jnp.sum(h * as_ref[...][None, :], axis=1), zv])
        d_ref[...] = jnp.concatenate(
            [jnp.sum(h * ad_ref[...][None, :], axis=1), zv])

    return pl.pallas_call(
        body,
        out_shape=(jax.ShapeDtypeStruct((_NPAD, _F), jnp.float32),
                   jax.ShapeDtypeStruct((_NPAD,), jnp.float32),
                   jax.ShapeDtypeStruct((_NPAD,), jnp.float32)),
    )(x, W, att_s, att_d)


def _tc_combine(acc, den, b, W, att_s, att_d):
    def body(acc_ref, den_ref, b_ref, w_ref, as_ref, ad_ref,
             h_ref, s_ref, d_ref):
        a = acc_ref[0] + acc_ref[1]
        dn = den_ref[0] + den_ref[1]
        h1 = a[:_N] / (dn[:_N, None] + _EPS) + b_ref[...][None, :]
        h1 = jnp.maximum(h1, _NEG_ACT * h1)
        h2 = jnp.dot(h1, w_ref[...], preferred_element_type=jnp.float32)
        zf = jnp.zeros((_NPAD - _N, _F), jnp.float32)
        h_ref[...] = jnp.concatenate([h2, zf], axis=0)
        zv = jnp.zeros((_NPAD - _N,), jnp.float32)
        s_ref[...] = jnp.concatenate(
            [jnp.sum(h2 * as_ref[...][None, :], axis=1), zv])
        d_ref[...] = jnp.concatenate(
            [jnp.sum(h2 * ad_ref[...][None, :], axis=1), zv])

    return pl.pallas_call(
        body,
        out_shape=(jax.ShapeDtypeStruct((_NPAD, _F), jnp.float32),
                   jax.ShapeDtypeStruct((_NPAD,), jnp.float32),
                   jax.ShapeDtypeStruct((_NPAD,), jnp.float32)),
    )(acc, den, b, W, att_s, att_d)


def _tc_final(acc, den, b):
    def body(acc_ref, den_ref, b_ref, out_ref):
        a = acc_ref[0] + acc_ref[1]
        dn = den_ref[0] + den_ref[1]
        out_ref[...] = a[:_N] / (dn[:_N, None] + _EPS) + b_ref[...][None, :]

    return pl.pallas_call(
        body,
        out_shape=jax.ShapeDtypeStruct((_N, _F), jnp.float32),
    )(acc, den, b)


def kernel(x, edge_index, W1, att_src1, att_dst1, bias1,
           W2, att_src2, att_dst2, bias2):
    src = edge_index[0].astype(jnp.int32)
    dst = edge_index[1].astype(jnp.int32)
    pad = jnp.full((_EPAD - _E,), _NPAD - 1, jnp.int32)
    srcp = jnp.concatenate([src, pad]).reshape(_NCHTOT, _CHUNK)
    dstp = jnp.concatenate([dst, pad]).reshape(_NCHTOT, _CHUNK)

    h1, s1, d1 = _tc_dense(x, W1, att_src1, att_dst1)
    acc1, den1 = _sc_edge(srcp, dstp, s1, d1, h1)
    h2, s2, d2 = _tc_combine(acc1, den1, bias1, W2, att_src2, att_dst2)
    acc2, den2 = _sc_edge(srcp, dstp, s2, d2, h2)
    return _tc_final(acc2, den2, bias2)


# speedup vs baseline: 1.0677x; 1.0362x over previous
import jax
import jax.numpy as jnp
from jax import lax
from jax.experimental import pallas as pl
from jax.experimental.pallas import tpu as pltpu
from jax.experimental.pallas import tpu_sc as plsc

_N = 10000
_F = 128
_NPAD = 10240
_E = 320000
_NTILES = 32
_CHUNK = 112
_NBUF = 3
_S0 = 23
_S1 = 7
_NCHTOT = 6 * (_S0 + _S1) * 16
_EPAD = _NCHTOT * _CHUNK
_ROWS_PT = _NPAD // 16
_NEG_ATT = 0.2
_NEG_ACT = 0.01
_EPS = 1e-16


def _sc_edge_body(src_hbm, dst_hbm, asrc_hbm, adst_hbm, h_hbm,
                  acc_out, den_out,
                  idx_s, idx_d, sv, dv, wv, rows,
                  isem, lsem, gsem, ssem, dsem, acc_sh, den_sh):
    cid = lax.axis_index("c")
    sid = lax.axis_index("s")
    nsuper = jnp.where(cid == 0, _S0, _S1)
    chunk0 = jnp.where(cid == 0, sid * (6 * _S0),
                       16 * 6 * _S0 + sid * (6 * _S1))

    zero16 = jnp.zeros((16,), jnp.float32)

    def _zrow(i, carry):
        for j in range(_F // 16):
            rows[0, i, pl.ds(j * 16, 16)] = zero16
        return carry

    lax.fori_loop(0, _CHUNK, _zrow, 0)
    for g in range(_CHUNK // 16):
        wv[0, pl.ds(g * 16, 16)] = zero16
    row0 = sid * _ROWS_PT
    for k in range(_ROWS_PT // _CHUNK):
        pltpu.sync_copy(rows.at[0], acc_sh.at[pl.ds(row0 + k * _CHUNK,
                                                    _CHUNK)])
        pltpu.sync_copy(wv.at[0], den_sh.at[pl.ds(row0 + k * _CHUNK, _CHUNK)])
    rem = _ROWS_PT % _CHUNK
    if rem:
        tail = row0 + (_ROWS_PT // _CHUNK) * _CHUNK
        pltpu.sync_copy(rows.at[0, pl.ds(0, rem)],
                        acc_sh.at[pl.ds(tail, rem)])
        pltpu.sync_copy(wv.at[0, pl.ds(0, rem)], den_sh.at[pl.ds(tail, rem)])
    plsc.subcore_barrier()

    def _issue_idx(i, b, islot):
        pltpu.async_copy(src_hbm.at[chunk0 + i], idx_s.at[islot], isem[b])
        pltpu.async_copy(dst_hbm.at[chunk0 + i], idx_d.at[islot], isem[b])

    def _wait_idx(i, b, islot):
        pltpu.make_async_copy(src_hbm.at[chunk0 + i], idx_s.at[islot],
                              isem[b]).wait()
        pltpu.make_async_copy(dst_hbm.at[chunk0 + i], idx_d.at[islot],
                              isem[b]).wait()

    def _issue_logits(b, islot):
        pltpu.async_copy(asrc_hbm.at[idx_s.at[islot]], sv.at[b], lsem[b])
        pltpu.async_copy(adst_hbm.at[idx_d.at[islot]], dv.at[b], lsem[b])

    def _wait_logits(b, islot):
        pltpu.make_async_copy(asrc_hbm.at[idx_s.at[islot]], sv.at[b],
                              lsem[b]).wait()
        pltpu.make_async_copy(adst_hbm.at[idx_d.at[islot]], dv.at[b],
                              lsem[b]).wait()

    def _issue_rows(b, islot):
        pltpu.async_copy(h_hbm.at[idx_s.at[islot]], rows.at[b], gsem[b])

    def _wait_rows(b, islot):
        pltpu.make_async_copy(h_hbm.at[idx_s.at[islot]], rows.at[b],
                              gsem[b]).wait()

    def _issue_scat_rows(b, islot):
        pltpu.async_copy(rows.at[b], acc_sh.at[idx_d.at[islot]], ssem[b],
                         add=True)

    def _wait_scat_rows(b, islot):
        pltpu.make_async_copy(rows.at[b], acc_sh.at[idx_d.at[islot]],
                              ssem[b]).wait()

    def _issue_scat_den(b, islot):
        pltpu.async_copy(wv.at[b], den_sh.at[idx_d.at[islot]], dsem[b],
                         add=True)

    def _wait_scat_den(b, islot):
        pltpu.make_async_copy(wv.at[b], den_sh.at[idx_d.at[islot]],
                              dsem[b]).wait()

    def _process(b, islot):
        _wait_logits(b, islot)
        for g in range(_CHUNK // 16):
            e16 = sv[b, pl.ds(g * 16, 16)] + dv[b, pl.ds(g * 16, 16)]
            e16 = jnp.maximum(e16, _NEG_ATT * e16)
            wv[b, pl.ds(g * 16, 16)] = jnp.exp(e16)
        _issue_scat_den(b, islot)
        _wait_rows(b, islot)
        b16 = jnp.full((16,), b, jnp.int32)

        @plsc.parallel_loop(0, _CHUNK, unroll=4)
        def _scale(e):
            w16 = plsc.load_gather(wv, [b16, jnp.full((16,), e, jnp.int32)])
            for j in range(_F // 16):
                rows[b, e, pl.ds(j * 16, 16)] = (
                    rows[b, e, pl.ds(j * 16, 16)] * w16)

        _issue_scat_rows(b, islot)

    for b in range(_NBUF):
        pltpu.sync_copy(src_hbm.at[chunk0 + b], idx_s.at[b])
        pltpu.sync_copy(dst_hbm.at[chunk0 + b], idx_d.at[b])
    for b in range(_NBUF):
        _issue_idx(_NBUF + b, b, _NBUF + b)
        _issue_logits(b, b)
        _issue_rows(b, b)

    def _super(s, carry):
        for r in range(2):
            base = 6 * s + 3 * r

            def _top():
                for b in range(_NBUF):
                    islot = 3 * r + b
                    islot_n = (islot + 3) % 6
                    _wait_scat_rows(b, islot_n)
                    _wait_scat_den(b, islot_n)
                    _wait_idx(base + b, b, islot)
                    _issue_logits(b, islot)
                    _issue_rows(b, islot)

            def _issue_next_idx():
                for b in range(_NBUF):
                    islot_n = (3 * r + b + 3) % 6
                    _issue_idx(base + b + 3, b, islot_n)

            if r == 0:
                @pl.when(s > 0)
                def _():
                    _top()
                    _issue_next_idx()
            else:
                _top()

                @pl.when(s + 1 < nsuper)
                def _():
                    _issue_next_idx()

            for b in range(_NBUF):
                _process(b, 3 * r + b)
        return carry

    lax.fori_loop(0, nsuper, _super, 0)

    for b in range(_NBUF):
        _wait_scat_rows(b, 3 + b)
        _wait_scat_den(b, 3 + b)

    plsc.subcore_barrier()
    pltpu.sync_copy(acc_sh.at[pl.ds(row0, _ROWS_PT)],
                    acc_out.at[cid, pl.ds(row0, _ROWS_PT)])
    pltpu.sync_copy(den_sh.at[pl.ds(row0, _ROWS_PT)],
                    den_out.at[cid, pl.ds(row0, _ROWS_PT)])


def _sc_edge(src2d, dst2d, a_src, a_dst, h):
    kern = pl.kernel(
        _sc_edge_body,
        out_type=(jax.ShapeDtypeStruct((2, _NPAD, _F), jnp.float32),
                  jax.ShapeDtypeStruct((2, _NPAD), jnp.float32)),
        mesh=plsc.VectorSubcoreMesh(core_axis_name="c", subcore_axis_name="s"),
        compiler_params=pltpu.CompilerParams(needs_layout_passes=False),
        scratch_types=[
            pltpu.VMEM((2 * _NBUF, _CHUNK), jnp.int32),
            pltpu.VMEM((2 * _NBUF, _CHUNK), jnp.int32),
            pltpu.VMEM((_NBUF, _CHUNK), jnp.float32),
            pltpu.VMEM((_NBUF, _CHUNK), jnp.float32),
            pltpu.VMEM((_NBUF, _CHUNK), jnp.float32),
            pltpu.VMEM((_NBUF, _CHUNK, _F), jnp.float32),
            [pltpu.SemaphoreType.DMA] * _NBUF,
            [pltpu.SemaphoreType.DMA] * _NBUF,
            [pltpu.SemaphoreType.DMA] * _NBUF,
            [pltpu.SemaphoreType.DMA] * _NBUF,
            [pltpu.SemaphoreType.DMA] * _NBUF,
            pltpu.VMEM_SHARED((_NPAD, _F), jnp.float32),
            pltpu.VMEM_SHARED((_NPAD,), jnp.float32),
        ],
    )
    return kern(src2d, dst2d, a_src, a_dst, h)


def _tc_dense(x, W, att_s, att_d):
    def body(x_ref, w_ref, as_ref, ad_ref, h_ref, s_ref, d_ref):
        h = jnp.dot(x_ref[...], w_ref[...],
                    preferred_element_type=jnp.float32)
        zf = jnp.zeros((_NPAD - _N, _F), jnp.float32)
        h_ref[...] = jnp.concatenate([h, zf], axis=0)
        zv = jnp.zeros((_NPAD - _N,), jnp.float32)
        s_ref[...] = jnp.concatenate(
            [jnp.sum(h * as_ref[...][None, :], axis=1), zv])
        d_ref[...] = jnp.concatenate(
            [jnp.sum(h * ad_ref[...][None, :], axis=1), zv])

    return pl.pallas_call(
        body,
        out_shape=(jax.ShapeDtypeStruct((_NPAD, _F), jnp.float32),
                   jax.ShapeDtypeStruct((_NPAD,), jnp.float32),
                   jax.ShapeDtypeStruct((_NPAD,), jnp.float32)),
    )(x, W, att_s, att_d)


def _tc_combine(acc, den, b, W, att_s, att_d):
    def body(acc_ref, den_ref, b_ref, w_ref, as_ref, ad_ref,
             h_ref, s_ref, d_ref):
        a = acc_ref[0] + acc_ref[1]
        dn = den_ref[0] + den_ref[1]
        h1 = a[:_N] / (dn[:_N, None] + _EPS) + b_ref[...][None, :]
        h1 = jnp.maximum(h1, _NEG_ACT * h1)
        h2 = jnp.dot(h1, w_ref[...], preferred_element_type=jnp.float32)
        zf = jnp.zeros((_NPAD - _N, _F), jnp.float32)
        h_ref[...] = jnp.concatenate([h2, zf], axis=0)
        zv = jnp.zeros((_NPAD - _N,), jnp.float32)
        s_ref[...] = jnp.concatenate(
            [jnp.sum(h2 * as_ref[...][None, :], axis=1), zv])
        d_ref[...] = jnp.concatenate(
            [jnp.sum(h2 * ad_ref[...][None, :], axis=1), zv])

    return pl.pallas_call(
        body,
        out_shape=(jax.ShapeDtypeStruct((_NPAD, _F), jnp.float32),
                   jax.ShapeDtypeStruct((_NPAD,), jnp.float32),
                   jax.ShapeDtypeStruct((_NPAD,), jnp.float32)),
    )(acc, den, b, W, att_s, att_d)


def _tc_final(acc, den, b):
    def body(acc_ref, den_ref, b_ref, out_ref):
        a = acc_ref[0] + acc_ref[1]
        dn = den_ref[0] + den_ref[1]
        out_ref[...] = a[:_N] / (dn[:_N, None] + _EPS) + b_ref[...][None, :]

    return pl.pallas_call(
        body,
        out_shape=jax.ShapeDtypeStruct((_N, _F), jnp.float32),
    )(acc, den, b)


def kernel(x, edge_index, W1, att_src1, att_dst1, bias1,
           W2, att_src2, att_dst2, bias2):
    src = edge_index[0].astype(jnp.int32)
    dst = edge_index[1].astype(jnp.int32)
    pad = jnp.full((_EPAD - _E,), _NPAD - 1, jnp.int32)
    srcp = jnp.concatenate([src, pad]).reshape(_NCHTOT, _CHUNK)
    dstp = jnp.concatenate([dst, pad]).reshape(_NCHTOT, _CHUNK)

    h1, s1, d1 = _tc_dense(x, W1, att_src1, att_dst1)
    acc1, den1 = _sc_edge(srcp, dstp, s1, d1, h1)
    h2, s2, d2 = _tc_combine(acc1, den1, bias1, W2, att_src2, att_dst2)
    acc2, den2 = _sc_edge(srcp, dstp, s2, d2, h2)
    return _tc_final(acc2, den2, bias2)
